# Initial kernel scaffold; baseline (speedup 1.0000x reference)
import jax
import jax.numpy as jnp
from jax.experimental import pallas as pl


def kernel(packed_weight_qvals, weight_scales, weight_zeros, x):
    raise NotImplementedError("write your pallas kernel here")


# speedup vs baseline: 4.9070x; 4.9070x over previous
import functools

import jax
import jax.numpy as jnp
from jax import lax
from jax.experimental import pallas as pl
from jax.experimental.pallas import tpu as pltpu
from jax.experimental.pallas import tpu_sc as plsc

DIM = 64
NGROUPS = 2
WORDS = DIM // 4
NW = 32
SUB = 128


def _dequant_gather(qtab, aux, idx2, n_flat):
    rows_per_w = n_flat // NW
    chunk = 1024
    nchunks = rows_per_w // chunk
    nsub = chunk // SUB
    mesh = plsc.VectorSubcoreMesh(core_axis_name="c", subcore_axis_name="s")

    @functools.partial(
        pl.kernel,
        mesh=mesh,
        out_type=jax.ShapeDtypeStruct((n_flat, DIM), jnp.float32),
        compiler_params=pltpu.CompilerParams(needs_layout_passes=False, use_tc_tiling_on_sc=False),
        scratch_types=[
            pltpu.VMEM((nsub, SUB), jnp.int32),
            pltpu.VMEM((chunk, WORDS), jnp.int32),
            pltpu.VMEM((chunk, 16), jnp.float32),
            pltpu.VMEM((chunk, DIM), jnp.float32),
            pltpu.SemaphoreType.DMA,
        ],
    )
    def body(qtab_ref, aux_ref, idx_ref, out_ref, idx_v, q_v, a_v, out_v, sem):
        wid = lax.axis_index("s") * 2 + lax.axis_index("c")
        lanes = lax.iota(jnp.int32, 16)
        scol = lanes // 8
        zcol = scol + NGROUPS
        ccols = [lanes * 4 + k for k in range(4)]

        for c in range(nchunks):
            base = wid * rows_per_w + c * chunk
            ib = wid * (rows_per_w // SUB) + c * nsub
            pltpu.sync_copy(idx_ref.at[pl.ds(ib, nsub)], idx_v)
            copies = []
            for j in range(nsub):
                copies.append(pltpu.async_copy(
                    qtab_ref.at[idx_v.at[j]],
                    q_v.at[pl.ds(j * SUB, SUB)], sem))
                copies.append(pltpu.async_copy(
                    aux_ref.at[idx_v.at[j]],
                    a_v.at[pl.ds(j * SUB, SUB)], sem))
            for cp in copies:
                cp.wait()

            def row_body(r, carry):
                rsp = jnp.full((16,), r, jnp.int32)
                qw = plsc.load_gather(q_v, [rsp, lanes])
                sv = plsc.load_gather(a_v, [rsp, scol])
                zv = plsc.load_gather(a_v, [rsp, zcol])
                for k in range(4):
                    pk = (qw << (24 - 8 * k)) >> 24 if k < 3 else qw >> 24
                    res = (pk.astype(jnp.float32) - zv) * sv
                    plsc.store_scatter(out_v, [rsp, ccols[k]], res)
                return carry

            lax.fori_loop(0, chunk, row_body, 0)
            pltpu.sync_copy(out_v, out_ref.at[pl.ds(base, chunk)])

    return body(qtab, aux, idx2)


def kernel(packed_weight_qvals, weight_scales, weight_zeros, x):
    V, D = packed_weight_qvals.shape
    qtab = lax.bitcast_convert_type(
        packed_weight_qvals.reshape(V, WORDS, 4), jnp.int32)
    aux = jnp.concatenate(
        [weight_scales, weight_zeros.astype(jnp.float32),
         jnp.zeros((V, 12), jnp.float32)], axis=1)
    flat = x.reshape(-1).astype(jnp.int32)
    n_flat = flat.shape[0]
    idx2 = flat.reshape(n_flat // SUB, SUB)
    out = _dequant_gather(qtab, aux, idx2, n_flat)
    return out.reshape(*x.shape, D)
